# Initial kernel scaffold; baseline (speedup 1.0000x reference)
import jax
import jax.numpy as jnp
from jax.experimental import pallas as pl


def kernel(x, alpha):
    raise NotImplementedError("write your pallas kernel here")


# speedup vs baseline: 1.2975x; 1.2975x over previous
import functools

import jax
import jax.numpy as jnp
from jax import lax
from jax.experimental import pallas as pl
from jax.experimental.pallas import tpu as pltpu
from jax.experimental.pallas import tpu_sc as plsc

B, C, T, K = 32, 256, 4096, 64
BETA = 10.0

NC = 2
NS = 16
NW = NC * NS
ROWS = B * K
RPW = ROWS // NW
CHUNK = 8
NCHUNK = RPW // CHUNK


def _alpha_body(a_ref, p_ref, w_ref, g_ref):
    a = a_ref[...]
    s = a * (1.0 / BETA)
    m = jnp.max(s, axis=1, keepdims=True)
    e = jnp.exp(s - m)
    p = e / jnp.sum(e, axis=1, keepdims=True)

    am = jnp.max(a, axis=1, keepdims=True)
    iota_c = lax.broadcasted_iota(jnp.int32, (K, C), 1)
    idx = jnp.min(jnp.where(a == am, iota_c, C), axis=1)
    hard = (iota_c == idx[:, None]).astype(a.dtype)

    p_ref[...] = p
    w_ref[...] = p + (hard - p)
    bi = lax.broadcasted_iota(jnp.int32, (B, K), 0)
    g_ref[...] = bi * C + idx[None, :]


_alpha_call = pl.pallas_call(
    _alpha_body,
    out_shape=(
        jax.ShapeDtypeStruct((K, C), jnp.float32),
        jax.ShapeDtypeStruct((K, C), jnp.float32),
        jax.ShapeDtypeStruct((B, K), jnp.int32),
    ),
)

_mesh = plsc.VectorSubcoreMesh(core_axis_name="c", subcore_axis_name="s")


@functools.partial(
    pl.kernel,
    mesh=_mesh,
    out_type=jax.ShapeDtypeStruct((ROWS, T), jnp.float32),
    scratch_types=[
        pltpu.VMEM((NCHUNK, CHUNK), jnp.int32),
        pltpu.VMEM((CHUNK, T), jnp.float32),
        pltpu.VMEM((CHUNK, T), jnp.float32),
        pltpu.SemaphoreType.DMA,
        pltpu.SemaphoreType.DMA,
    ],
)
def _gather(x_hbm, g_hbm, z_hbm, idx_v, buf0, buf1, sem0, sem1):
    wid = lax.axis_index("s") * NC + lax.axis_index("c")
    base = wid * RPW
    pltpu.sync_copy(g_hbm.at[wid], idx_v)

    bufs = (buf0, buf1)
    sems = (sem0, sem1)
    copies = [
        pltpu.async_copy(x_hbm.at[idx_v.at[c]], bufs[c], sems[c])
        for c in range(2)
    ]
    for c in range(NCHUNK):
        copies[c].wait()
        pltpu.sync_copy(bufs[c % 2], z_hbm.at[pl.ds(base + c * CHUNK, CHUNK)])
        if c + 2 < NCHUNK:
            copies.append(
                pltpu.async_copy(
                    x_hbm.at[idx_v.at[c + 2]], bufs[c % 2], sems[c % 2]
                )
            )


def kernel(x, alpha):
    p, w, g = _alpha_call(alpha)
    x2 = x.reshape(B * C, T)
    g3 = g.reshape(NW, NCHUNK, CHUNK)
    z = _gather(x2, g3)
    return (z.reshape(B, 1, K, T), w, p)


# speedup vs baseline: 1.3444x; 1.0361x over previous
import functools

import jax
import jax.numpy as jnp
from jax import lax
from jax.experimental import pallas as pl
from jax.experimental.pallas import tpu as pltpu
from jax.experimental.pallas import tpu_sc as plsc

B, C, T, K = 32, 256, 4096, 64
BETA = 10.0

NC = 2
NS = 16
NW = NC * NS
ROWS = B * K
RPW = ROWS // NW
CHUNK = 8
NCHUNK = RPW // CHUNK


def _alpha_body(a_ref, p_ref, w_ref, g_ref):
    a = a_ref[...]
    s = a * (1.0 / BETA)
    m = jnp.max(s, axis=1, keepdims=True)
    e = jnp.exp(s - m)
    p = e / jnp.sum(e, axis=1, keepdims=True)

    am = jnp.max(a, axis=1, keepdims=True)
    iota_c = lax.broadcasted_iota(jnp.int32, (K, C), 1)
    idx = jnp.min(jnp.where(a == am, iota_c, C), axis=1)
    hard = (iota_c == idx[:, None]).astype(a.dtype)

    p_ref[...] = p
    w_ref[...] = p + (hard - p)
    bi = lax.broadcasted_iota(jnp.int32, (B, K), 0)
    g_ref[...] = bi * C + idx[None, :]


_alpha_call = pl.pallas_call(
    _alpha_body,
    out_shape=(
        jax.ShapeDtypeStruct((K, C), jnp.float32),
        jax.ShapeDtypeStruct((K, C), jnp.float32),
        jax.ShapeDtypeStruct((B, K), jnp.int32),
    ),
)

_mesh = plsc.VectorSubcoreMesh(core_axis_name="c", subcore_axis_name="s")

NBUF = 3


@functools.partial(
    pl.kernel,
    mesh=_mesh,
    out_type=jax.ShapeDtypeStruct((ROWS, T), jnp.float32),
    scratch_types=[
        pltpu.VMEM((RPW,), jnp.int32),
        pltpu.VMEM((CHUNK, T), jnp.float32),
        pltpu.VMEM((CHUNK, T), jnp.float32),
        pltpu.VMEM((CHUNK, T), jnp.float32),
        pltpu.SemaphoreType.DMA,
        pltpu.SemaphoreType.DMA,
        pltpu.SemaphoreType.DMA,
        pltpu.SemaphoreType.DMA,
        pltpu.SemaphoreType.DMA,
        pltpu.SemaphoreType.DMA,
    ],
)
def _gather(x_hbm, g_hbm, z_hbm, idx_v, buf0, buf1, buf2,
            gs0, gs1, gs2, ss0, ss1, ss2):
    wid = lax.axis_index("s") * NC + lax.axis_index("c")
    base = wid * RPW
    pltpu.sync_copy(g_hbm.at[wid], idx_v)

    bufs = (buf0, buf1, buf2)
    gsems = (gs0, gs1, gs2)
    ssems = (ss0, ss1, ss2)
    gcp, scp = {}, {}
    for c in range(NBUF):
        gcp[c] = pltpu.async_copy(
            x_hbm.at[idx_v.at[pl.ds(c * CHUNK, CHUNK)]],
            bufs[c % NBUF], gsems[c % NBUF])
    for c in range(NCHUNK):
        gcp[c].wait()
        scp[c] = pltpu.async_copy(
            bufs[c % NBUF],
            z_hbm.at[pl.ds(base + c * CHUNK, CHUNK)], ssems[c % NBUF])
        if c + NBUF < NCHUNK:
            scp[c].wait()
            gcp[c + NBUF] = pltpu.async_copy(
                x_hbm.at[idx_v.at[pl.ds((c + NBUF) * CHUNK, CHUNK)]],
                bufs[c % NBUF], gsems[c % NBUF])
    for c in range(max(0, NCHUNK - NBUF), NCHUNK):
        scp[c].wait()


def kernel(x, alpha):
    p, w, g = _alpha_call(alpha)
    x2 = x.reshape(B * C, T)
    z = _gather(x2, g)
    return (z.reshape(B, 1, K, T), w, p)
